# baseline (speedup 1.0000x reference)
import functools

import jax
import jax.numpy as jnp
from jax import lax
from jax.experimental import pallas as pl
from jax.experimental.pallas import tpu as pltpu
from jax.experimental.pallas import tpu_sc as plsc

_N = 10000
_E = 160000
_DIN = 128
_UNITS = 64
_B = 2
_INPUT_SIZE = _DIN + _UNITS
_NUM_MATRICES = 5
_C = _INPUT_SIZE * _B
_NW = 32
_COLS_PER_W = _C // _NW
_COLS_PER_PASS = 6
_PASSES = _COLS_PER_W // _COLS_PER_PASS
_ECH = 1600
_NCHUNK = _E // _ECH
_LANES = 16
_PW = _COLS_PER_PASS * _N


def _diffusion_body(x0_hbm, src_hbm, dst_hbm, v1_hbm, v2_hbm,
                    m1_hbm, m2_hbm, m3_hbm, m4_hbm, xb, yb, gb, sb, vb):
    wid = lax.axis_index("c") * 16 + lax.axis_index("s")

    def accumulate(x_hbm, g_hbm, s_hbm, v_hbm, base):
        pltpu.sync_copy(x_hbm.at[pl.ds(base, _PW)], xb)

        def _zero(i, _):
            yb[pl.ds(i * _LANES, _LANES)] = jnp.zeros((_LANES,), jnp.float32)
            return 0
        lax.fori_loop(0, _PW // _LANES, _zero, 0)

        def _chunk(ch, _):
            e0 = pl.multiple_of(ch * _ECH, 8)
            pltpu.sync_copy(g_hbm.at[pl.ds(e0, _ECH)], gb)
            pltpu.sync_copy(s_hbm.at[pl.ds(e0, _ECH)], sb)
            pltpu.sync_copy(v_hbm.at[pl.ds(e0, _ECH)], vb)

            def _grp(j, _):
                gi = gb[pl.ds(j * _LANES, _LANES)]
                si = sb[pl.ds(j * _LANES, _LANES)]
                vv = vb[pl.ds(j * _LANES, _LANES)]
                for c in range(_COLS_PER_PASS):
                    off = jnp.int32(c * _N)
                    xv = plsc.load_gather(xb, [gi + off])
                    plsc.addupdate_scatter(yb, [si + off], xv * vv)
                return 0
            lax.fori_loop(0, _ECH // _LANES, _grp, 0)
            return 0
        lax.fori_loop(0, _NCHUNK, _chunk, 0)

    def axpy_from(z_hbm, base):
        pltpu.sync_copy(z_hbm.at[pl.ds(base, _PW)], xb)

        def _ax(i, _):
            sl = pl.ds(i * _LANES, _LANES)
            yb[sl] = 2.0 * yb[sl] - xb[sl]
            return 0
        lax.fori_loop(0, _PW // _LANES, _ax, 0)

    for p in range(_PASSES):
        base = (wid * _COLS_PER_W + p * _COLS_PER_PASS) * _N
        accumulate(x0_hbm, src_hbm, dst_hbm, v1_hbm, base)
        pltpu.sync_copy(yb, m1_hbm.at[pl.ds(base, _PW)])
        accumulate(m1_hbm, src_hbm, dst_hbm, v1_hbm, base)
        axpy_from(x0_hbm, base)
        pltpu.sync_copy(yb, m2_hbm.at[pl.ds(base, _PW)])
        accumulate(m1_hbm, dst_hbm, src_hbm, v2_hbm, base)
        pltpu.sync_copy(yb, m3_hbm.at[pl.ds(base, _PW)])
        accumulate(m3_hbm, dst_hbm, src_hbm, v2_hbm, base)
        axpy_from(m1_hbm, base)
        pltpu.sync_copy(yb, m4_hbm.at[pl.ds(base, _PW)])


def _diffusion_sc(x0T, src, dst, v1, v2):
    mesh = plsc.VectorSubcoreMesh(core_axis_name="c", subcore_axis_name="s")
    sds = jax.ShapeDtypeStruct((_C * _N,), jnp.float32)
    ms = pl.kernel(
        _diffusion_body,
        out_type=[sds, sds, sds, sds],
        mesh=mesh,
        compiler_params=pltpu.CompilerParams(
            needs_layout_passes=False, use_tc_tiling_on_sc=False
        ),
        scratch_types=[
            pltpu.VMEM((_PW,), jnp.float32),
            pltpu.VMEM((_PW,), jnp.float32),
            pltpu.VMEM((_ECH,), jnp.int32),
            pltpu.VMEM((_ECH,), jnp.int32),
            pltpu.VMEM((_ECH,), jnp.float32),
        ],
    )(x0T.reshape(_C * _N), src, dst, v1, v2)
    return [m.reshape(_C, _N) for m in ms]


_BM = 2000
_K = _INPUT_SIZE * _NUM_MATRICES


def _gates_body(x_ref, w_ref, b_ref, hx_ref, rh_ref, u_ref):
    acc = jnp.dot(x_ref[...], w_ref[...], preferred_element_type=jnp.float32)
    val = jax.nn.sigmoid(acc + b_ref[...])
    r = val[:, :_UNITS]
    u = val[:, _UNITS:]
    rh_ref[...] = r * hx_ref[...]
    u_ref[...] = u


def _gates_matmul(xcat, w, b, hx2d):
    grid = (xcat.shape[0] // _BM,)
    return pl.pallas_call(
        _gates_body,
        grid=grid,
        in_specs=[
            pl.BlockSpec((_BM, _K), lambda i: (i, 0)),
            pl.BlockSpec((_K, 2 * _UNITS), lambda i: (0, 0)),
            pl.BlockSpec((1, 2 * _UNITS), lambda i: (0, 0)),
            pl.BlockSpec((_BM, _UNITS), lambda i: (i, 0)),
        ],
        out_specs=[
            pl.BlockSpec((_BM, _UNITS), lambda i: (i, 0)),
            pl.BlockSpec((_BM, _UNITS), lambda i: (i, 0)),
        ],
        out_shape=[
            jax.ShapeDtypeStruct((xcat.shape[0], _UNITS), jnp.float32),
            jax.ShapeDtypeStruct((xcat.shape[0], _UNITS), jnp.float32),
        ],
    )(xcat, w, b, hx2d)


def _cand_body(x_ref, w_ref, b_ref, u_ref, hx_ref, out_ref):
    acc = jnp.dot(x_ref[...], w_ref[...], preferred_element_type=jnp.float32)
    c = jnp.tanh(acc + b_ref[...])
    u = u_ref[...]
    out_ref[...] = u * hx_ref[...] + (1.0 - u) * c


def _cand_matmul(xcat, w, b, u2d, hx2d):
    grid = (xcat.shape[0] // _BM,)
    return pl.pallas_call(
        _cand_body,
        grid=grid,
        in_specs=[
            pl.BlockSpec((_BM, _K), lambda i: (i, 0)),
            pl.BlockSpec((_K, _UNITS), lambda i: (0, 0)),
            pl.BlockSpec((1, _UNITS), lambda i: (0, 0)),
            pl.BlockSpec((_BM, _UNITS), lambda i: (i, 0)),
            pl.BlockSpec((_BM, _UNITS), lambda i: (i, 0)),
        ],
        out_specs=pl.BlockSpec((_BM, _UNITS), lambda i: (i, 0)),
        out_shape=jax.ShapeDtypeStruct((xcat.shape[0], _UNITS), jnp.float32),
    )(xcat, w, b, u2d, hx2d)


def _diffuse(x0T, src, dst, v1, v2):
    m1, m2, m3, m4 = _diffusion_sc(x0T, src, dst, v1, v2)
    return [x0T, m1, m2, m3, m4]


def _assemble_xcat(ms):
    a = jnp.stack(ms, axis=0).reshape(_NUM_MATRICES, _INPUT_SIZE, _B, _N)
    return jnp.transpose(a, (2, 3, 1, 0)).reshape(_B * _N, _INPUT_SIZE * _NUM_MATRICES)


def _build_x0T(inputs, state2d):
    xi = inputs.reshape(_B, _N, _DIN)
    xs = state2d.reshape(_B, _N, _UNITS)
    x = jnp.concatenate([xi, xs], axis=2)
    return jnp.transpose(x, (2, 0, 1)).reshape(_C, _N)


def kernel(inputs, hx, edge_index, v1, v2, W_gates, b_gates, W_cand, b_cand):
    src = edge_index[0]
    dst = edge_index[1]
    hx2d = hx.reshape(_B, _N, _UNITS).reshape(_B * _N, _UNITS)

    x0T = _build_x0T(inputs, hx2d)
    xcat = _assemble_xcat(_diffuse(x0T, src, dst, v1, v2))
    rh, u = _gates_matmul(xcat, W_gates, b_gates.reshape(1, -1), hx2d)

    x0T2 = _build_x0T(inputs, rh)
    xcat2 = _assemble_xcat(_diffuse(x0T2, src, dst, v1, v2))
    new2d = _cand_matmul(xcat2, W_cand, b_cand.reshape(1, -1), u, hx2d)

    return new2d.reshape(_B, _N * _UNITS)


# speedup vs baseline: 1.6730x; 1.6730x over previous
import functools

import jax
import jax.numpy as jnp
from jax import lax
from jax.experimental import pallas as pl
from jax.experimental.pallas import tpu as pltpu
from jax.experimental.pallas import tpu_sc as plsc

_N = 10000
_E = 160000
_DIN = 128
_UNITS = 64
_B = 2
_INPUT_SIZE = _DIN + _UNITS
_NUM_MATRICES = 5
_C = _INPUT_SIZE * _B
_NW = 32
_COLS_PER_W = _C // _NW
_COLS_PER_PASS = 6
_PASSES = _COLS_PER_W // _COLS_PER_PASS
_ECH = 1600
_NCHUNK = _E // _ECH
_LANES = 16
_PW = _COLS_PER_PASS * _N


def _diffusion_body(x0_hbm, src_hbm, dst_hbm, v1_hbm, v2_hbm,
                    m1_hbm, m2_hbm, m3_hbm, m4_hbm, xb, yb, gb, sb, vb):
    wid = lax.axis_index("c") * 16 + lax.axis_index("s")

    def accumulate(x_hbm, g_hbm, s_hbm, v_hbm, base):
        pltpu.sync_copy(x_hbm.at[pl.ds(base, _PW)], xb)

        @plsc.parallel_loop(0, _PW // _LANES, unroll=8)
        def _zero(i):
            yb[pl.ds(i * _LANES, _LANES)] = jnp.zeros((_LANES,), jnp.float32)

        def _chunk(ch, _):
            e0 = pl.multiple_of(ch * _ECH, 8)
            pltpu.sync_copy(g_hbm.at[pl.ds(e0, _ECH)], gb)
            pltpu.sync_copy(s_hbm.at[pl.ds(e0, _ECH)], sb)
            pltpu.sync_copy(v_hbm.at[pl.ds(e0, _ECH)], vb)

            @plsc.parallel_loop(0, _ECH // _LANES, unroll=4)
            def _grp(j):
                gi = gb[pl.ds(j * _LANES, _LANES)]
                si = sb[pl.ds(j * _LANES, _LANES)]
                vv = vb[pl.ds(j * _LANES, _LANES)]
                for c in range(_COLS_PER_PASS):
                    off = jnp.int32(c * _N)
                    xv = plsc.load_gather(xb, [gi + off])
                    plsc.addupdate_scatter(yb, [si + off], xv * vv)
            return 0
        lax.fori_loop(0, _NCHUNK, _chunk, 0)

    def axpy_from(z_hbm, base):
        pltpu.sync_copy(z_hbm.at[pl.ds(base, _PW)], xb)

        @plsc.parallel_loop(0, _PW // _LANES, unroll=8)
        def _ax(i):
            sl = pl.ds(i * _LANES, _LANES)
            yb[sl] = 2.0 * yb[sl] - xb[sl]

    for p in range(_PASSES):
        base = (wid * _COLS_PER_W + p * _COLS_PER_PASS) * _N
        accumulate(x0_hbm, src_hbm, dst_hbm, v1_hbm, base)
        pltpu.sync_copy(yb, m1_hbm.at[pl.ds(base, _PW)])
        accumulate(m1_hbm, src_hbm, dst_hbm, v1_hbm, base)
        axpy_from(x0_hbm, base)
        pltpu.sync_copy(yb, m2_hbm.at[pl.ds(base, _PW)])
        accumulate(m1_hbm, dst_hbm, src_hbm, v2_hbm, base)
        pltpu.sync_copy(yb, m3_hbm.at[pl.ds(base, _PW)])
        accumulate(m3_hbm, dst_hbm, src_hbm, v2_hbm, base)
        axpy_from(m1_hbm, base)
        pltpu.sync_copy(yb, m4_hbm.at[pl.ds(base, _PW)])


def _diffusion_sc(x0T, src, dst, v1, v2):
    mesh = plsc.VectorSubcoreMesh(core_axis_name="c", subcore_axis_name="s")
    sds = jax.ShapeDtypeStruct((_C * _N,), jnp.float32)
    ms = pl.kernel(
        _diffusion_body,
        out_type=[sds, sds, sds, sds],
        mesh=mesh,
        compiler_params=pltpu.CompilerParams(
            needs_layout_passes=False, use_tc_tiling_on_sc=False
        ),
        scratch_types=[
            pltpu.VMEM((_PW,), jnp.float32),
            pltpu.VMEM((_PW,), jnp.float32),
            pltpu.VMEM((_ECH,), jnp.int32),
            pltpu.VMEM((_ECH,), jnp.int32),
            pltpu.VMEM((_ECH,), jnp.float32),
        ],
    )(x0T.reshape(_C * _N), src, dst, v1, v2)
    return [m.reshape(_C, _N) for m in ms]


_BM = 2000
_K = _INPUT_SIZE * _NUM_MATRICES


def _gates_body(x_ref, w_ref, b_ref, hx_ref, rh_ref, u_ref):
    acc = jnp.dot(x_ref[...], w_ref[...], preferred_element_type=jnp.float32)
    val = jax.nn.sigmoid(acc + b_ref[...])
    r = val[:, :_UNITS]
    u = val[:, _UNITS:]
    rh_ref[...] = r * hx_ref[...]
    u_ref[...] = u


def _gates_matmul(xcat, w, b, hx2d):
    grid = (xcat.shape[0] // _BM,)
    return pl.pallas_call(
        _gates_body,
        grid=grid,
        in_specs=[
            pl.BlockSpec((_BM, _K), lambda i: (i, 0)),
            pl.BlockSpec((_K, 2 * _UNITS), lambda i: (0, 0)),
            pl.BlockSpec((1, 2 * _UNITS), lambda i: (0, 0)),
            pl.BlockSpec((_BM, _UNITS), lambda i: (i, 0)),
        ],
        out_specs=[
            pl.BlockSpec((_BM, _UNITS), lambda i: (i, 0)),
            pl.BlockSpec((_BM, _UNITS), lambda i: (i, 0)),
        ],
        out_shape=[
            jax.ShapeDtypeStruct((xcat.shape[0], _UNITS), jnp.float32),
            jax.ShapeDtypeStruct((xcat.shape[0], _UNITS), jnp.float32),
        ],
    )(xcat, w, b, hx2d)


def _cand_body(x_ref, w_ref, b_ref, u_ref, hx_ref, out_ref):
    acc = jnp.dot(x_ref[...], w_ref[...], preferred_element_type=jnp.float32)
    c = jnp.tanh(acc + b_ref[...])
    u = u_ref[...]
    out_ref[...] = u * hx_ref[...] + (1.0 - u) * c


def _cand_matmul(xcat, w, b, u2d, hx2d):
    grid = (xcat.shape[0] // _BM,)
    return pl.pallas_call(
        _cand_body,
        grid=grid,
        in_specs=[
            pl.BlockSpec((_BM, _K), lambda i: (i, 0)),
            pl.BlockSpec((_K, _UNITS), lambda i: (0, 0)),
            pl.BlockSpec((1, _UNITS), lambda i: (0, 0)),
            pl.BlockSpec((_BM, _UNITS), lambda i: (i, 0)),
            pl.BlockSpec((_BM, _UNITS), lambda i: (i, 0)),
        ],
        out_specs=pl.BlockSpec((_BM, _UNITS), lambda i: (i, 0)),
        out_shape=jax.ShapeDtypeStruct((xcat.shape[0], _UNITS), jnp.float32),
    )(xcat, w, b, u2d, hx2d)


def _diffuse(x0T, src, dst, v1, v2):
    m1, m2, m3, m4 = _diffusion_sc(x0T, src, dst, v1, v2)
    return [x0T, m1, m2, m3, m4]


def _assemble_xcat(ms):
    a = jnp.stack(ms, axis=0).reshape(_NUM_MATRICES, _INPUT_SIZE, _B, _N)
    return jnp.transpose(a, (2, 3, 1, 0)).reshape(_B * _N, _INPUT_SIZE * _NUM_MATRICES)


def _build_x0T(inputs, state2d):
    xi = inputs.reshape(_B, _N, _DIN)
    xs = state2d.reshape(_B, _N, _UNITS)
    x = jnp.concatenate([xi, xs], axis=2)
    return jnp.transpose(x, (2, 0, 1)).reshape(_C, _N)


def kernel(inputs, hx, edge_index, v1, v2, W_gates, b_gates, W_cand, b_cand):
    src = edge_index[0]
    dst = edge_index[1]
    hx2d = hx.reshape(_B, _N, _UNITS).reshape(_B * _N, _UNITS)

    x0T = _build_x0T(inputs, hx2d)
    xcat = _assemble_xcat(_diffuse(x0T, src, dst, v1, v2))
    rh, u = _gates_matmul(xcat, W_gates, b_gates.reshape(1, -1), hx2d)

    x0T2 = _build_x0T(inputs, rh)
    xcat2 = _assemble_xcat(_diffuse(x0T2, src, dst, v1, v2))
    new2d = _cand_matmul(xcat2, W_cand, b_cand.reshape(1, -1), u, hx2d)

    return new2d.reshape(_B, _N * _UNITS)


# speedup vs baseline: 2.0086x; 1.2006x over previous
import functools

import jax
import jax.numpy as jnp
from jax import lax
from jax.experimental import pallas as pl
from jax.experimental.pallas import tpu as pltpu
from jax.experimental.pallas import tpu_sc as plsc

_N = 10000
_E = 160000
_DIN = 128
_UNITS = 64
_B = 2
_INPUT_SIZE = _DIN + _UNITS
_NUM_MATRICES = 5
_C = _INPUT_SIZE * _B
_NW = 32
_COLS_PER_W = _C // _NW
_COLS_PER_PASS = 6
_PASSES = _COLS_PER_W // _COLS_PER_PASS
_ECH = 3200
_NCHUNK = _E // _ECH
_LANES = 16
_PW = _COLS_PER_PASS * _N


def _diffusion_body(x0_hbm, src_hbm, dst_hbm, v1_hbm, v2_hbm,
                    m1_hbm, m2_hbm, m3_hbm, m4_hbm, xb, yb, gb, sb, vb):
    wid = lax.axis_index("c") * 16 + lax.axis_index("s")

    def accumulate(x_hbm, g_hbm, s_hbm, v_hbm, base):
        pltpu.sync_copy(x_hbm.at[pl.ds(base, _PW)], xb)

        @plsc.parallel_loop(0, _PW // _LANES, unroll=8)
        def _zero(i):
            yb[pl.ds(i * _LANES, _LANES)] = jnp.zeros((_LANES,), jnp.float32)

        def _chunk(ch, _):
            e0 = pl.multiple_of(ch * _ECH, 8)
            pltpu.sync_copy(g_hbm.at[pl.ds(e0, _ECH)], gb)
            pltpu.sync_copy(s_hbm.at[pl.ds(e0, _ECH)], sb)
            pltpu.sync_copy(v_hbm.at[pl.ds(e0, _ECH)], vb)

            @plsc.parallel_loop(0, _ECH // _LANES, unroll=4)
            def _grp(j):
                gi = gb[pl.ds(j * _LANES, _LANES)]
                si = sb[pl.ds(j * _LANES, _LANES)]
                vv = vb[pl.ds(j * _LANES, _LANES)]
                for c in range(_COLS_PER_PASS):
                    off = jnp.int32(c * _N)
                    xv = plsc.load_gather(xb, [gi + off])
                    plsc.addupdate_scatter(yb, [si + off], xv * vv)
            return 0
        lax.fori_loop(0, _NCHUNK, _chunk, 0)

    def axpy_from(z_hbm, base):
        pltpu.sync_copy(z_hbm.at[pl.ds(base, _PW)], xb)

        @plsc.parallel_loop(0, _PW // _LANES, unroll=8)
        def _ax(i):
            sl = pl.ds(i * _LANES, _LANES)
            yb[sl] = 2.0 * yb[sl] - xb[sl]

    for p in range(_PASSES):
        base = (wid * _COLS_PER_W + p * _COLS_PER_PASS) * _N
        accumulate(x0_hbm, src_hbm, dst_hbm, v1_hbm, base)
        pltpu.sync_copy(yb, m1_hbm.at[pl.ds(base, _PW)])
        accumulate(m1_hbm, src_hbm, dst_hbm, v1_hbm, base)
        axpy_from(x0_hbm, base)
        pltpu.sync_copy(yb, m2_hbm.at[pl.ds(base, _PW)])
        accumulate(m1_hbm, dst_hbm, src_hbm, v2_hbm, base)
        pltpu.sync_copy(yb, m3_hbm.at[pl.ds(base, _PW)])
        accumulate(m3_hbm, dst_hbm, src_hbm, v2_hbm, base)
        axpy_from(m1_hbm, base)
        pltpu.sync_copy(yb, m4_hbm.at[pl.ds(base, _PW)])


def _diffusion_sc(x0T, src, dst, v1, v2):
    mesh = plsc.VectorSubcoreMesh(core_axis_name="c", subcore_axis_name="s")
    sds = jax.ShapeDtypeStruct((_C * _N,), jnp.float32)
    ms = pl.kernel(
        _diffusion_body,
        out_type=[sds, sds, sds, sds],
        mesh=mesh,
        compiler_params=pltpu.CompilerParams(
            needs_layout_passes=False, use_tc_tiling_on_sc=False
        ),
        scratch_types=[
            pltpu.VMEM((_PW,), jnp.float32),
            pltpu.VMEM((_PW,), jnp.float32),
            pltpu.VMEM((_ECH,), jnp.int32),
            pltpu.VMEM((_ECH,), jnp.int32),
            pltpu.VMEM((_ECH,), jnp.float32),
        ],
    )(x0T.reshape(_C * _N), src, dst, v1, v2)
    return [m.reshape(_C, _N) for m in ms]


_BM = 2000
_K = _INPUT_SIZE * _NUM_MATRICES


def _gates_body(x_ref, w_ref, b_ref, hx_ref, rh_ref, u_ref):
    acc = jnp.dot(x_ref[...], w_ref[...], preferred_element_type=jnp.float32)
    val = jax.nn.sigmoid(acc + b_ref[...])
    r = val[:, :_UNITS]
    u = val[:, _UNITS:]
    rh_ref[...] = r * hx_ref[...]
    u_ref[...] = u


def _gates_matmul(xcat, w, b, hx2d):
    grid = (xcat.shape[0] // _BM,)
    return pl.pallas_call(
        _gates_body,
        grid=grid,
        in_specs=[
            pl.BlockSpec((_BM, _K), lambda i: (i, 0)),
            pl.BlockSpec((_K, 2 * _UNITS), lambda i: (0, 0)),
            pl.BlockSpec((1, 2 * _UNITS), lambda i: (0, 0)),
            pl.BlockSpec((_BM, _UNITS), lambda i: (i, 0)),
        ],
        out_specs=[
            pl.BlockSpec((_BM, _UNITS), lambda i: (i, 0)),
            pl.BlockSpec((_BM, _UNITS), lambda i: (i, 0)),
        ],
        out_shape=[
            jax.ShapeDtypeStruct((xcat.shape[0], _UNITS), jnp.float32),
            jax.ShapeDtypeStruct((xcat.shape[0], _UNITS), jnp.float32),
        ],
    )(xcat, w, b, hx2d)


def _cand_body(x_ref, w_ref, b_ref, u_ref, hx_ref, out_ref):
    acc = jnp.dot(x_ref[...], w_ref[...], preferred_element_type=jnp.float32)
    c = jnp.tanh(acc + b_ref[...])
    u = u_ref[...]
    out_ref[...] = u * hx_ref[...] + (1.0 - u) * c


def _cand_matmul(xcat, w, b, u2d, hx2d):
    grid = (xcat.shape[0] // _BM,)
    return pl.pallas_call(
        _cand_body,
        grid=grid,
        in_specs=[
            pl.BlockSpec((_BM, _K), lambda i: (i, 0)),
            pl.BlockSpec((_K, _UNITS), lambda i: (0, 0)),
            pl.BlockSpec((1, _UNITS), lambda i: (0, 0)),
            pl.BlockSpec((_BM, _UNITS), lambda i: (i, 0)),
            pl.BlockSpec((_BM, _UNITS), lambda i: (i, 0)),
        ],
        out_specs=pl.BlockSpec((_BM, _UNITS), lambda i: (i, 0)),
        out_shape=jax.ShapeDtypeStruct((xcat.shape[0], _UNITS), jnp.float32),
    )(xcat, w, b, u2d, hx2d)


def _diffuse(x0T, src, dst, v1, v2):
    m1, m2, m3, m4 = _diffusion_sc(x0T, src, dst, v1, v2)
    return [x0T, m1, m2, m3, m4]


def _assemble_xcat(ms):
    a = jnp.stack(ms, axis=0).reshape(_NUM_MATRICES, _INPUT_SIZE, _B, _N)
    return jnp.transpose(a, (2, 3, 1, 0)).reshape(_B * _N, _INPUT_SIZE * _NUM_MATRICES)


def _build_x0T(inputs, state2d):
    xi = inputs.reshape(_B, _N, _DIN)
    xs = state2d.reshape(_B, _N, _UNITS)
    x = jnp.concatenate([xi, xs], axis=2)
    return jnp.transpose(x, (2, 0, 1)).reshape(_C, _N)


def kernel(inputs, hx, edge_index, v1, v2, W_gates, b_gates, W_cand, b_cand):
    src = edge_index[0]
    dst = edge_index[1]
    hx2d = hx.reshape(_B, _N, _UNITS).reshape(_B * _N, _UNITS)

    x0T = _build_x0T(inputs, hx2d)
    xcat = _assemble_xcat(_diffuse(x0T, src, dst, v1, v2))
    rh, u = _gates_matmul(xcat, W_gates, b_gates.reshape(1, -1), hx2d)

    x0T2 = _build_x0T(inputs, rh)
    xcat2 = _assemble_xcat(_diffuse(x0T2, src, dst, v1, v2))
    new2d = _cand_matmul(xcat2, W_cand, b_cand.reshape(1, -1), u, hx2d)

    return new2d.reshape(_B, _N * _UNITS)


# speedup vs baseline: 2.7391x; 1.3637x over previous
import functools

import jax
import jax.numpy as jnp
from jax import lax
from jax.experimental import pallas as pl
from jax.experimental.pallas import tpu as pltpu
from jax.experimental.pallas import tpu_sc as plsc

_N = 10000
_E = 160000
_DIN = 128
_UNITS = 64
_B = 2
_INPUT_SIZE = _DIN + _UNITS
_NUM_MATRICES = 5
_C = _INPUT_SIZE * _B
_NW = 32
_COLS_PER_W = _C // _NW
_COLS_PER_PASS = 6
_PASSES = _COLS_PER_W // _COLS_PER_PASS
_ECH = 1600
_NCHUNK = _E // _ECH
_LANES = 16
_PW = _COLS_PER_PASS * _N


def _diffusion_body(x0_hbm, src_hbm, dst_hbm, v1_hbm, v2_hbm,
                    m1_hbm, m2_hbm, m3_hbm, m4_hbm, xb, yb,
                    gb0, sb0, vb0, gb1, sb1, vb1, sem0, sem1):
    wid = lax.axis_index("c") * 16 + lax.axis_index("s")
    bufs = ((gb0, sb0, vb0, sem0), (gb1, sb1, vb1, sem1))

    def accumulate(x_hbm, g_hbm, s_hbm, v_hbm, base):
        pltpu.sync_copy(x_hbm.at[pl.ds(base, _PW)], xb)

        @plsc.parallel_loop(0, _PW // _LANES, unroll=8)
        def _zero(i):
            yb[pl.ds(i * _LANES, _LANES)] = jnp.zeros((_LANES,), jnp.float32)

        def issue(ch, k):
            gbuf, sbuf, vbuf, sem = bufs[k]
            e0 = pl.multiple_of(ch * _ECH, 8)
            pltpu.async_copy(g_hbm.at[pl.ds(e0, _ECH)], gbuf, sem)
            pltpu.async_copy(s_hbm.at[pl.ds(e0, _ECH)], sbuf, sem)
            pltpu.async_copy(v_hbm.at[pl.ds(e0, _ECH)], vbuf, sem)

        def drain(k):
            gbuf, sbuf, vbuf, sem = bufs[k]
            pltpu.make_async_copy(g_hbm.at[pl.ds(0, _ECH)], gbuf, sem).wait()
            pltpu.make_async_copy(s_hbm.at[pl.ds(0, _ECH)], sbuf, sem).wait()
            pltpu.make_async_copy(v_hbm.at[pl.ds(0, _ECH)], vbuf, sem).wait()

        def compute(k):
            gbuf, sbuf, vbuf, _ = bufs[k]

            @plsc.parallel_loop(0, _ECH // _LANES, unroll=4)
            def _grp(j):
                gi = gbuf[pl.ds(j * _LANES, _LANES)]
                si = sbuf[pl.ds(j * _LANES, _LANES)]
                vv = vbuf[pl.ds(j * _LANES, _LANES)]
                for c in range(_COLS_PER_PASS):
                    off = jnp.int32(c * _N)
                    xv = plsc.load_gather(xb, [gi + off])
                    plsc.addupdate_scatter(yb, [si + off], xv * vv)

        issue(jnp.int32(0), 0)
        issue(jnp.int32(1), 1)

        def _pair(i, _):
            drain(0)
            compute(0)
            issue(jnp.minimum(2 * i + 2, _NCHUNK - 2), 0)
            drain(1)
            compute(1)
            issue(jnp.minimum(2 * i + 3, _NCHUNK - 1), 1)
            return 0
        lax.fori_loop(0, _NCHUNK // 2, _pair, 0)
        drain(0)
        drain(1)

    def axpy_from(z_hbm, base):
        pltpu.sync_copy(z_hbm.at[pl.ds(base, _PW)], xb)

        @plsc.parallel_loop(0, _PW // _LANES, unroll=8)
        def _ax(i):
            sl = pl.ds(i * _LANES, _LANES)
            yb[sl] = 2.0 * yb[sl] - xb[sl]

    for p in range(_PASSES):
        base = (wid * _COLS_PER_W + p * _COLS_PER_PASS) * _N
        accumulate(x0_hbm, src_hbm, dst_hbm, v1_hbm, base)
        pltpu.sync_copy(yb, m1_hbm.at[pl.ds(base, _PW)])
        accumulate(m1_hbm, src_hbm, dst_hbm, v1_hbm, base)
        axpy_from(x0_hbm, base)
        pltpu.sync_copy(yb, m2_hbm.at[pl.ds(base, _PW)])
        accumulate(m1_hbm, dst_hbm, src_hbm, v2_hbm, base)
        pltpu.sync_copy(yb, m3_hbm.at[pl.ds(base, _PW)])
        accumulate(m3_hbm, dst_hbm, src_hbm, v2_hbm, base)
        axpy_from(m1_hbm, base)
        pltpu.sync_copy(yb, m4_hbm.at[pl.ds(base, _PW)])


def _diffusion_sc(x0T, src, dst, v1, v2):
    mesh = plsc.VectorSubcoreMesh(core_axis_name="c", subcore_axis_name="s")
    sds = jax.ShapeDtypeStruct((_C * _N,), jnp.float32)
    ms = pl.kernel(
        _diffusion_body,
        out_type=[sds, sds, sds, sds],
        mesh=mesh,
        compiler_params=pltpu.CompilerParams(
            needs_layout_passes=False, use_tc_tiling_on_sc=False
        ),
        scratch_types=[
            pltpu.VMEM((_PW,), jnp.float32),
            pltpu.VMEM((_PW,), jnp.float32),
            pltpu.VMEM((_ECH,), jnp.int32),
            pltpu.VMEM((_ECH,), jnp.int32),
            pltpu.VMEM((_ECH,), jnp.float32),
            pltpu.VMEM((_ECH,), jnp.int32),
            pltpu.VMEM((_ECH,), jnp.int32),
            pltpu.VMEM((_ECH,), jnp.float32),
            pltpu.SemaphoreType.DMA,
            pltpu.SemaphoreType.DMA,
        ],
    )(x0T.reshape(_C * _N), src, dst, v1, v2)
    return [m.reshape(_C, _N) for m in ms]


_BM = 2000
_K = _INPUT_SIZE * _NUM_MATRICES


def _gates_body(x_ref, w_ref, b_ref, hx_ref, rh_ref, u_ref):
    acc = jnp.dot(x_ref[...], w_ref[...], preferred_element_type=jnp.float32)
    val = jax.nn.sigmoid(acc + b_ref[...])
    r = val[:, :_UNITS]
    u = val[:, _UNITS:]
    rh_ref[...] = r * hx_ref[...]
    u_ref[...] = u


def _gates_matmul(xcat, w, b, hx2d):
    grid = (xcat.shape[0] // _BM,)
    return pl.pallas_call(
        _gates_body,
        grid=grid,
        in_specs=[
            pl.BlockSpec((_BM, _K), lambda i: (i, 0)),
            pl.BlockSpec((_K, 2 * _UNITS), lambda i: (0, 0)),
            pl.BlockSpec((1, 2 * _UNITS), lambda i: (0, 0)),
            pl.BlockSpec((_BM, _UNITS), lambda i: (i, 0)),
        ],
        out_specs=[
            pl.BlockSpec((_BM, _UNITS), lambda i: (i, 0)),
            pl.BlockSpec((_BM, _UNITS), lambda i: (i, 0)),
        ],
        out_shape=[
            jax.ShapeDtypeStruct((xcat.shape[0], _UNITS), jnp.float32),
            jax.ShapeDtypeStruct((xcat.shape[0], _UNITS), jnp.float32),
        ],
    )(xcat, w, b, hx2d)


def _cand_body(x_ref, w_ref, b_ref, u_ref, hx_ref, out_ref):
    acc = jnp.dot(x_ref[...], w_ref[...], preferred_element_type=jnp.float32)
    c = jnp.tanh(acc + b_ref[...])
    u = u_ref[...]
    out_ref[...] = u * hx_ref[...] + (1.0 - u) * c


def _cand_matmul(xcat, w, b, u2d, hx2d):
    grid = (xcat.shape[0] // _BM,)
    return pl.pallas_call(
        _cand_body,
        grid=grid,
        in_specs=[
            pl.BlockSpec((_BM, _K), lambda i: (i, 0)),
            pl.BlockSpec((_K, _UNITS), lambda i: (0, 0)),
            pl.BlockSpec((1, _UNITS), lambda i: (0, 0)),
            pl.BlockSpec((_BM, _UNITS), lambda i: (i, 0)),
            pl.BlockSpec((_BM, _UNITS), lambda i: (i, 0)),
        ],
        out_specs=pl.BlockSpec((_BM, _UNITS), lambda i: (i, 0)),
        out_shape=jax.ShapeDtypeStruct((xcat.shape[0], _UNITS), jnp.float32),
    )(xcat, w, b, u2d, hx2d)


def _diffuse(x0T, src, dst, v1, v2):
    m1, m2, m3, m4 = _diffusion_sc(x0T, src, dst, v1, v2)
    return [x0T, m1, m2, m3, m4]


def _assemble_xcat(ms):
    a = jnp.stack(ms, axis=0).reshape(_NUM_MATRICES, _INPUT_SIZE, _B, _N)
    return jnp.transpose(a, (2, 3, 1, 0)).reshape(_B * _N, _INPUT_SIZE * _NUM_MATRICES)


def _build_x0T(inputs, state2d):
    xi = inputs.reshape(_B, _N, _DIN)
    xs = state2d.reshape(_B, _N, _UNITS)
    x = jnp.concatenate([xi, xs], axis=2)
    return jnp.transpose(x, (2, 0, 1)).reshape(_C, _N)


def kernel(inputs, hx, edge_index, v1, v2, W_gates, b_gates, W_cand, b_cand):
    src = edge_index[0]
    dst = edge_index[1]
    hx2d = hx.reshape(_B, _N, _UNITS).reshape(_B * _N, _UNITS)

    x0T = _build_x0T(inputs, hx2d)
    xcat = _assemble_xcat(_diffuse(x0T, src, dst, v1, v2))
    rh, u = _gates_matmul(xcat, W_gates, b_gates.reshape(1, -1), hx2d)

    x0T2 = _build_x0T(inputs, rh)
    xcat2 = _assemble_xcat(_diffuse(x0T2, src, dst, v1, v2))
    new2d = _cand_matmul(xcat2, W_cand, b_cand.reshape(1, -1), u, hx2d)

    return new2d.reshape(_B, _N * _UNITS)
